# baseline (speedup 1.0000x reference)
import jax
import jax.numpy as jnp
from jax import lax
from jax.experimental import pallas as pl
from jax.experimental.pallas import tpu as pltpu
from jax.experimental.pallas import tpu_sc as plsc

_NUM_CORES = 2
_NUM_SUBCORES = 16
_NUM_WORKERS = _NUM_CORES * _NUM_SUBCORES
_CHUNK_ROWS = 24
_NBUF = 2


def _copy_body(table_hbm, out_hbm, bufs, ld_sems, st_sems):
    sid = lax.axis_index("s")
    wid = sid * _NUM_CORES + lax.axis_index("c")
    rows = table_hbm.shape[0] // _NUM_WORKERS
    base = wid * rows

    offs = []
    o = 0
    while o < rows:
        c = min(_CHUNK_ROWS, rows - o)
        offs.append((o, c))
        o += c
    nchunks = len(offs)

    def load(g, b):
        o, c = offs[g]
        return pltpu.async_copy(
            table_hbm.at[pl.ds(base + o, c), :],
            bufs.at[sid, b, pl.ds(0, c)],
            ld_sems.at[b],
        )

    def store(g, b):
        o, c = offs[g]
        return pltpu.async_copy(
            bufs.at[sid, b, pl.ds(0, c)],
            out_hbm.at[0, pl.ds(base + o, c), :],
            st_sems.at[b],
        )

    loads = {}
    stores = {}
    for g in range(min(_NBUF, nchunks)):
        loads[g] = load(g, g)
    for g in range(nchunks):
        b = g % _NBUF
        loads.pop(g).wait()
        stores[g] = store(g, b)
        j = g - 1
        if j >= 0 and j + _NBUF < nchunks:
            stores.pop(j).wait()
            loads[j + _NBUF] = load(j + _NBUF, j % _NBUF)
    for g in sorted(stores):
        stores.pop(g).wait()


@jax.jit
def kernel(x, pos_embedding):
    seq_len = x.shape[1]
    d_model = pos_embedding.shape[1]
    mesh = plsc.VectorSubcoreMesh(core_axis_name="c", subcore_axis_name="s")
    fn = pl.kernel(
        _copy_body,
        out_type=jax.ShapeDtypeStruct((1, seq_len, d_model), jnp.float32),
        mesh=mesh,
        scratch_types=[
            pltpu.VMEM_SHARED(
                (_NUM_SUBCORES, _NBUF, _CHUNK_ROWS, d_model), jnp.float32
            ),
            pltpu.SemaphoreType.DMA((_NBUF,)),
            pltpu.SemaphoreType.DMA((_NBUF,)),
        ],
    )
    return fn(pos_embedding[:seq_len])
